# baseline (speedup 1.0000x reference)
import functools

import jax
import jax.numpy as jnp
from jax.experimental import pallas as pl
from jax.experimental.pallas import tpu as pltpu
from jax.sharding import Mesh, PartitionSpec as P
from jax import shard_map


def _body(x_ref, w_ref, a_ref, o_ref, support_ref):
    @pl.when(pl.program_id(0) == 0)
    def _():
        support_ref[...] = jnp.dot(
            x_ref[...], w_ref[...], preferred_element_type=jnp.float32
        )

    o_ref[...] = jnp.dot(
        a_ref[...], support_ref[...], preferred_element_type=jnp.float32
    )


def _graph_conv(input_tensor, adj_mat, weights, block_rows=400):
    n_rows = adj_mat.shape[0]
    n, f_in = input_tensor.shape
    f_out = weights.shape[1]
    grid = pl.cdiv(n_rows, block_rows)
    return pl.pallas_call(
        _body,
        grid=(grid,),
        in_specs=[
            pl.BlockSpec((n, f_in), lambda i: (0, 0)),
            pl.BlockSpec((f_in, f_out), lambda i: (0, 0)),
            pl.BlockSpec((block_rows, n), lambda i: (i, 0)),
        ],
        out_specs=pl.BlockSpec((block_rows, f_out), lambda i: (i, 0)),
        out_shape=jax.ShapeDtypeStruct((n_rows, f_out), jnp.float32),
        scratch_shapes=[pltpu.VMEM((n, f_out), jnp.float32)],
        compiler_params=pltpu.CompilerParams(
            dimension_semantics=("arbitrary",),
        ),
    )(input_tensor, weights, adj_mat)


def kernel(input_tensor, adj_mat, kernel):
    devs = jax.devices()
    n = adj_mat.shape[0]
    ndev = len(devs)
    if ndev > 1 and n % ndev == 0 and (n // ndev) % 8 == 0:
        mesh = Mesh(devs, ("x",))
        sharded = shard_map(
            _graph_conv,
            mesh=mesh,
            in_specs=(P(None, None), P("x", None), P(None, None)),
            out_specs=P("x", None),
            check_vma=False,
        )
        return sharded(input_tensor, adj_mat, kernel)
    return _graph_conv(input_tensor, adj_mat, kernel)


# speedup vs baseline: 5.0773x; 5.0773x over previous
import functools

import jax
import jax.numpy as jnp
from jax.experimental import pallas as pl
from jax.experimental.pallas import tpu as pltpu


def _body(x_ref, w_ref, a_ref, o_ref, support_ref):
    @pl.when(pl.program_id(0) == 0)
    def _():
        support_ref[...] = jnp.dot(
            x_ref[...], w_ref[...], preferred_element_type=jnp.float32
        )

    o_ref[...] = jnp.dot(
        a_ref[...], support_ref[...], preferred_element_type=jnp.float32
    )


@functools.partial(jax.jit, static_argnames=("block_rows",))
def _graph_conv(input_tensor, adj_mat, weights, block_rows=624):
    n, f_in = input_tensor.shape
    f_out = weights.shape[1]
    grid = pl.cdiv(n, block_rows)
    return pl.pallas_call(
        _body,
        grid=(grid,),
        in_specs=[
            pl.BlockSpec((n, f_in), lambda i: (0, 0)),
            pl.BlockSpec((f_in, f_out), lambda i: (0, 0)),
            pl.BlockSpec((block_rows, n), lambda i: (i, 0)),
        ],
        out_specs=pl.BlockSpec((block_rows, f_out), lambda i: (i, 0)),
        out_shape=jax.ShapeDtypeStruct((n, f_out), jnp.float32),
        scratch_shapes=[pltpu.VMEM((n, f_out), jnp.float32)],
        compiler_params=pltpu.CompilerParams(
            dimension_semantics=("arbitrary",),
            vmem_limit_bytes=64 * 1024 * 1024,
        ),
    )(input_tensor, weights, adj_mat)


def kernel(input_tensor, adj_mat, kernel):
    return _graph_conv(input_tensor, adj_mat, kernel)


# speedup vs baseline: 5.3223x; 1.0483x over previous
import functools

import jax
import jax.numpy as jnp
from jax.experimental import pallas as pl
from jax.experimental.pallas import tpu as pltpu


def _body(x_ref, w_ref, a_ref, o_ref, support_ref):
    @pl.when(pl.program_id(0) == 0)
    def _():
        support_ref[...] = jnp.dot(
            x_ref[...], w_ref[...], preferred_element_type=jnp.float32
        )

    o_ref[...] = jnp.dot(
        a_ref[...], support_ref[...], preferred_element_type=jnp.float32
    )


@functools.partial(jax.jit, static_argnames=("block_rows",))
def _graph_conv(input_tensor, adj_mat, weights, block_rows=400):
    n, f_in = input_tensor.shape
    f_out = weights.shape[1]
    grid = pl.cdiv(n, block_rows)
    return pl.pallas_call(
        _body,
        grid=(grid,),
        in_specs=[
            pl.BlockSpec((n, f_in), lambda i: (0, 0)),
            pl.BlockSpec((f_in, f_out), lambda i: (0, 0)),
            pl.BlockSpec((block_rows, n), lambda i: (i, 0)),
        ],
        out_specs=pl.BlockSpec((block_rows, f_out), lambda i: (i, 0)),
        out_shape=jax.ShapeDtypeStruct((n, f_out), jnp.float32),
        scratch_shapes=[pltpu.VMEM((n, f_out), jnp.float32)],
        compiler_params=pltpu.CompilerParams(
            dimension_semantics=("arbitrary",),
        ),
    )(input_tensor, weights, adj_mat)


def kernel(input_tensor, adj_mat, kernel):
    return _graph_conv(input_tensor, adj_mat, kernel)
